# Initial kernel scaffold; baseline (speedup 1.0000x reference)
import jax
import jax.numpy as jnp
from jax.experimental import pallas as pl


def kernel(backstory_embedding, backstory_trace, chunk_embeddings, chunk_traces, W1, b1, W2, b2):
    raise NotImplementedError("write your pallas kernel here")


# speedup vs baseline: 1.5821x; 1.5821x over previous
import functools

import jax
import jax.numpy as jnp
from jax.experimental import pallas as pl
from jax.experimental.pallas import tpu as pltpu

N_EMBD = 768
NEURON_DIM = 512
TOP_K = 64
HIDDEN = 256
N_CHUNKS = 100000

BLK = 1000
GRID = N_CHUNKS // BLK


def _fused_kernel(bt_ref, be_ref, ct_blk_ref, ct_hbm, ce_hbm,
                  W1_ref, b1_ref, W2_ref, b2_ref,
                  score_out, idx_out,
                  sims_ref, emb_buf, tr_buf, idx_smem, sem_e, sem_t):
    i = pl.program_id(0)

    bt = bt_ref[...]
    bt_scale = 1.0 / (jnp.sqrt(jnp.sum(bt * bt)) + 1e-8)
    blk = ct_blk_ref[...]
    dots = jnp.dot(blk, bt.reshape(NEURON_DIM, 1),
                   preferred_element_type=jnp.float32)
    sumsq = jnp.dot(blk * blk, jnp.ones((NEURON_DIM, 1), jnp.float32),
                    preferred_element_type=jnp.float32)
    sims = (dots * bt_scale) / (jnp.sqrt(sumsq) + 1e-8)
    sims_ref[pl.ds(i, 1), :] = sims.reshape(1, BLK)

    @pl.when(i == GRID - 1)
    def _tail():
        lin = (jax.lax.broadcasted_iota(jnp.int32, (GRID, BLK), 0) * BLK
               + jax.lax.broadcasted_iota(jnp.int32, (GRID, BLK), 1))

        def extract(k, _):
            s = sims_ref[...]
            m = jnp.max(s)
            gid = jnp.min(jnp.where(s == m, lin, jnp.int32(N_CHUNKS)))
            idx_smem[k] = gid
            sims_ref[...] = jnp.where(lin == gid, -jnp.inf, s)
            return 0

        jax.lax.fori_loop(0, TOP_K, extract, 0)

        for k in range(TOP_K):
            idx = idx_smem[k]
            pltpu.make_async_copy(
                ce_hbm.at[pl.ds(idx, 1), :], emb_buf.at[pl.ds(k, 1), :],
                sem_e).start()
            pltpu.make_async_copy(
                ct_hbm.at[pl.ds(idx, 1), :], tr_buf.at[pl.ds(k, 1), :],
                sem_t).start()
        for k in range(TOP_K):
            idx = idx_smem[k]
            pltpu.make_async_copy(
                ce_hbm.at[pl.ds(idx, 1), :], emb_buf.at[pl.ds(k, 1), :],
                sem_e).wait()
            pltpu.make_async_copy(
                ct_hbm.at[pl.ds(idx, 1), :], tr_buf.at[pl.ds(k, 1), :],
                sem_t).wait()

        be = be_ref[...]
        cvec = (jnp.dot(be, W1_ref[N_EMBD:2 * N_EMBD, :],
                        preferred_element_type=jnp.float32)
                + jnp.dot(bt, W1_ref[2 * N_EMBD + NEURON_DIM:, :],
                          preferred_element_type=jnp.float32)
                + b1_ref[...])
        h = (jnp.dot(emb_buf[...], W1_ref[:N_EMBD, :],
                     preferred_element_type=jnp.float32)
             + jnp.dot(tr_buf[...],
                       W1_ref[2 * N_EMBD:2 * N_EMBD + NEURON_DIM, :],
                       preferred_element_type=jnp.float32)
             + cvec)
        h = jnp.maximum(h, 0.0)
        scores = jnp.dot(h, W2_ref[...],
                         preferred_element_type=jnp.float32) + b2_ref[0, 0]

        m = jnp.max(scores)
        kio = jax.lax.broadcasted_iota(jnp.int32, (TOP_K, 1), 0)
        karg = jnp.min(jnp.where(scores == m, kio, jnp.int32(TOP_K)))
        score_out[0] = m
        idx_out[0] = idx_smem[karg]


@jax.jit
def kernel(backstory_embedding, backstory_trace, chunk_embeddings,
           chunk_traces, W1, b1, W2, b2):
    bt = backstory_trace.reshape(1, NEURON_DIM)
    be = backstory_embedding.reshape(1, N_EMBD)
    score, idx = pl.pallas_call(
        _fused_kernel,
        grid=(GRID,),
        in_specs=[
            pl.BlockSpec((1, NEURON_DIM), lambda i: (0, 0)),
            pl.BlockSpec((1, N_EMBD), lambda i: (0, 0)),
            pl.BlockSpec((BLK, NEURON_DIM), lambda i: (i, 0)),
            pl.BlockSpec(memory_space=pltpu.MemorySpace.HBM),
            pl.BlockSpec(memory_space=pltpu.MemorySpace.HBM),
            pl.BlockSpec((2 * N_EMBD + 2 * NEURON_DIM, HIDDEN),
                         lambda i: (0, 0)),
            pl.BlockSpec((1, HIDDEN), lambda i: (0, 0)),
            pl.BlockSpec((HIDDEN, 1), lambda i: (0, 0)),
            pl.BlockSpec((1, 1), lambda i: (0, 0)),
        ],
        out_specs=[
            pl.BlockSpec(memory_space=pltpu.MemorySpace.SMEM),
            pl.BlockSpec(memory_space=pltpu.MemorySpace.SMEM),
        ],
        out_shape=[
            jax.ShapeDtypeStruct((1,), jnp.float32),
            jax.ShapeDtypeStruct((1,), jnp.int32),
        ],
        scratch_shapes=[
            pltpu.VMEM((GRID, BLK), jnp.float32),
            pltpu.VMEM((TOP_K, N_EMBD), jnp.float32),
            pltpu.VMEM((TOP_K, NEURON_DIM), jnp.float32),
            pltpu.SMEM((TOP_K,), jnp.int32),
            pltpu.SemaphoreType.DMA,
            pltpu.SemaphoreType.DMA,
        ],
    )(bt, be, chunk_traces, chunk_traces, chunk_embeddings, W1,
      b1.reshape(1, HIDDEN), W2, b2.reshape(1, 1))
    return score[0], idx[0]


# speedup vs baseline: 1.7527x; 1.1078x over previous
import functools

import jax
import jax.numpy as jnp
from jax.experimental import pallas as pl
from jax.experimental.pallas import tpu as pltpu

N_EMBD = 768
NEURON_DIM = 512
TOP_K = 64
HIDDEN = 256
N_CHUNKS = 100000

BLK = 1000
GRID = N_CHUNKS // BLK


def _fused_kernel(bt_ref, be_ref, ct_blk_ref, ct_hbm, ce_hbm,
                  W1_ref, b1_ref, W2_ref, b2_ref,
                  score_out, idx_out,
                  sims_ref, rowmax_ref, emb_buf, tr_buf, idx_smem,
                  sem_e, sem_t):
    i = pl.program_id(0)

    bt = bt_ref[...]
    bt_scale = 1.0 / (jnp.sqrt(jnp.sum(bt * bt)) + 1e-8)
    blk = ct_blk_ref[...]
    dn = (((1,), (1,)), ((), ()))
    dots = jax.lax.dot_general(bt, blk, dn,
                               preferred_element_type=jnp.float32)
    ones = jnp.ones((1, NEURON_DIM), jnp.float32)
    sumsq = jax.lax.dot_general(ones, blk * blk, dn,
                                preferred_element_type=jnp.float32)
    sims = (dots * bt_scale) / (jnp.sqrt(sumsq) + 1e-8)
    sims_ref[pl.ds(i, 1), :] = sims

    @pl.when(i == GRID - 1)
    def _tail():
        io_r = jax.lax.broadcasted_iota(jnp.int32, (GRID, 1), 0)
        io_c = jax.lax.broadcasted_iota(jnp.int32, (1, BLK), 1)
        rowmax_ref[...] = jnp.max(sims_ref[...], axis=1, keepdims=True)

        def extract(k, _):
            rm = rowmax_ref[...]
            g = jnp.max(rm)
            r = jnp.min(jnp.where(rm == g, io_r, jnp.int32(GRID)))
            row = sims_ref[pl.ds(r, 1), :]
            c = jnp.min(jnp.where(row == g, io_c, jnp.int32(BLK)))
            idx_smem[k] = r * BLK + c
            nrow = jnp.where(io_c == c, -jnp.inf, row)
            sims_ref[pl.ds(r, 1), :] = nrow
            rowmax_ref[pl.ds(r, 1), :] = jnp.max(nrow, axis=1, keepdims=True)
            return 0

        jax.lax.fori_loop(0, TOP_K, extract, 0)

        for k in range(TOP_K):
            idx = idx_smem[k]
            pltpu.make_async_copy(
                ce_hbm.at[pl.ds(idx, 1), :], emb_buf.at[pl.ds(k, 1), :],
                sem_e).start()
            pltpu.make_async_copy(
                ct_hbm.at[pl.ds(idx, 1), :], tr_buf.at[pl.ds(k, 1), :],
                sem_t).start()
        for k in range(TOP_K):
            idx = idx_smem[k]
            pltpu.make_async_copy(
                ce_hbm.at[pl.ds(idx, 1), :], emb_buf.at[pl.ds(k, 1), :],
                sem_e).wait()
            pltpu.make_async_copy(
                ct_hbm.at[pl.ds(idx, 1), :], tr_buf.at[pl.ds(k, 1), :],
                sem_t).wait()

        be = be_ref[...]
        cvec = (jnp.dot(be, W1_ref[N_EMBD:2 * N_EMBD, :],
                        preferred_element_type=jnp.float32)
                + jnp.dot(bt, W1_ref[2 * N_EMBD + NEURON_DIM:, :],
                          preferred_element_type=jnp.float32)
                + b1_ref[...])
        h = (jnp.dot(emb_buf[...], W1_ref[:N_EMBD, :],
                     preferred_element_type=jnp.float32)
             + jnp.dot(tr_buf[...],
                       W1_ref[2 * N_EMBD:2 * N_EMBD + NEURON_DIM, :],
                       preferred_element_type=jnp.float32)
             + cvec)
        h = jnp.maximum(h, 0.0)
        scores = jnp.dot(h, W2_ref[...],
                         preferred_element_type=jnp.float32) + b2_ref[0, 0]

        m = jnp.max(scores)
        kio = jax.lax.broadcasted_iota(jnp.int32, (TOP_K, 1), 0)
        karg = jnp.min(jnp.where(scores == m, kio, jnp.int32(TOP_K)))
        score_out[0] = m
        idx_out[0] = idx_smem[karg]


@jax.jit
def kernel(backstory_embedding, backstory_trace, chunk_embeddings,
           chunk_traces, W1, b1, W2, b2):
    bt = backstory_trace.reshape(1, NEURON_DIM)
    be = backstory_embedding.reshape(1, N_EMBD)
    score, idx = pl.pallas_call(
        _fused_kernel,
        grid=(GRID,),
        in_specs=[
            pl.BlockSpec((1, NEURON_DIM), lambda i: (0, 0)),
            pl.BlockSpec((1, N_EMBD), lambda i: (0, 0)),
            pl.BlockSpec((BLK, NEURON_DIM), lambda i: (i, 0)),
            pl.BlockSpec(memory_space=pltpu.MemorySpace.HBM),
            pl.BlockSpec(memory_space=pltpu.MemorySpace.HBM),
            pl.BlockSpec((2 * N_EMBD + 2 * NEURON_DIM, HIDDEN),
                         lambda i: (0, 0)),
            pl.BlockSpec((1, HIDDEN), lambda i: (0, 0)),
            pl.BlockSpec((HIDDEN, 1), lambda i: (0, 0)),
            pl.BlockSpec((1, 1), lambda i: (0, 0)),
        ],
        out_specs=[
            pl.BlockSpec(memory_space=pltpu.MemorySpace.SMEM),
            pl.BlockSpec(memory_space=pltpu.MemorySpace.SMEM),
        ],
        out_shape=[
            jax.ShapeDtypeStruct((1,), jnp.float32),
            jax.ShapeDtypeStruct((1,), jnp.int32),
        ],
        scratch_shapes=[
            pltpu.VMEM((GRID, BLK), jnp.float32),
            pltpu.VMEM((GRID, 1), jnp.float32),
            pltpu.VMEM((TOP_K, N_EMBD), jnp.float32),
            pltpu.VMEM((TOP_K, NEURON_DIM), jnp.float32),
            pltpu.SMEM((TOP_K,), jnp.int32),
            pltpu.SemaphoreType.DMA,
            pltpu.SemaphoreType.DMA,
        ],
    )(bt, be, chunk_traces, chunk_traces, chunk_embeddings, W1,
      b1.reshape(1, HIDDEN), W2, b2.reshape(1, 1))
    return score[0], idx[0]


# speedup vs baseline: 2.3524x; 1.3422x over previous
import functools

import jax
import jax.numpy as jnp
from jax.experimental import pallas as pl
from jax.experimental.pallas import tpu as pltpu

N_EMBD = 768
NEURON_DIM = 512
TOP_K = 64
HIDDEN = 256
N_CHUNKS = 100000

BLK = 1000
GRID = N_CHUNKS // BLK
NSTREAM = 4
STEPS = GRID // NSTREAM


def _fused_kernel(bt_ref, be_ref, ct0_ref, ct1_ref, ct2_ref, ct3_ref,
                  ct_hbm, ce_hbm,
                  W1_ref, b1_ref, W2_ref, b2_ref,
                  score_out, idx_out,
                  sims_ref, rowmax_ref, emb_buf, tr_buf, idx_smem,
                  sem_e, sem_t):
    i = pl.program_id(0)

    bt = bt_ref[...]
    bt_scale = 1.0 / (jnp.sqrt(jnp.sum(bt * bt)) + 1e-8)
    dn = (((1,), (1,)), ((), ()))
    ones = jnp.ones((1, NEURON_DIM), jnp.float32)
    for j, ct_ref in enumerate((ct0_ref, ct1_ref, ct2_ref, ct3_ref)):
        blk = ct_ref[...]
        dots = jax.lax.dot_general(bt, blk, dn,
                                   preferred_element_type=jnp.float32)
        sumsq = jax.lax.dot_general(ones, blk * blk, dn,
                                    preferred_element_type=jnp.float32)
        sims = (dots * bt_scale) / (jnp.sqrt(sumsq) + 1e-8)
        sims_ref[pl.ds(i + j * STEPS, 1), :] = sims

    @pl.when(i == STEPS - 1)
    def _tail():
        io_r = jax.lax.broadcasted_iota(jnp.int32, (GRID, 1), 0)
        io_c = jax.lax.broadcasted_iota(jnp.int32, (1, BLK), 1)
        rowmax_ref[...] = jnp.max(sims_ref[...], axis=1, keepdims=True)

        def extract(k, _):
            rm = rowmax_ref[...]
            g = jnp.max(rm)
            r = jnp.min(jnp.where(rm == g, io_r, jnp.int32(GRID)))
            row = sims_ref[pl.ds(r, 1), :]
            c = jnp.min(jnp.where(row == g, io_c, jnp.int32(BLK)))
            idx_smem[k] = r * BLK + c
            nrow = jnp.where(io_c == c, -jnp.inf, row)
            sims_ref[pl.ds(r, 1), :] = nrow
            rowmax_ref[pl.ds(r, 1), :] = jnp.max(nrow, axis=1, keepdims=True)
            return 0

        jax.lax.fori_loop(0, TOP_K, extract, 0)

        for k in range(TOP_K):
            idx = idx_smem[k]
            pltpu.make_async_copy(
                ce_hbm.at[pl.ds(idx, 1), :], emb_buf.at[pl.ds(k, 1), :],
                sem_e).start()
            pltpu.make_async_copy(
                ct_hbm.at[pl.ds(idx, 1), :], tr_buf.at[pl.ds(k, 1), :],
                sem_t).start()
        for k in range(TOP_K):
            idx = idx_smem[k]
            pltpu.make_async_copy(
                ce_hbm.at[pl.ds(idx, 1), :], emb_buf.at[pl.ds(k, 1), :],
                sem_e).wait()
            pltpu.make_async_copy(
                ct_hbm.at[pl.ds(idx, 1), :], tr_buf.at[pl.ds(k, 1), :],
                sem_t).wait()

        be = be_ref[...]
        cvec = (jnp.dot(be, W1_ref[N_EMBD:2 * N_EMBD, :],
                        preferred_element_type=jnp.float32)
                + jnp.dot(bt, W1_ref[2 * N_EMBD + NEURON_DIM:, :],
                          preferred_element_type=jnp.float32)
                + b1_ref[...])
        h = (jnp.dot(emb_buf[...], W1_ref[:N_EMBD, :],
                     preferred_element_type=jnp.float32)
             + jnp.dot(tr_buf[...],
                       W1_ref[2 * N_EMBD:2 * N_EMBD + NEURON_DIM, :],
                       preferred_element_type=jnp.float32)
             + cvec)
        h = jnp.maximum(h, 0.0)
        scores = jnp.dot(h, W2_ref[...],
                         preferred_element_type=jnp.float32) + b2_ref[0, 0]

        m = jnp.max(scores)
        kio = jax.lax.broadcasted_iota(jnp.int32, (TOP_K, 1), 0)
        karg = jnp.min(jnp.where(scores == m, kio, jnp.int32(TOP_K)))
        score_out[0] = m
        idx_out[0] = idx_smem[karg]


@jax.jit
def kernel(backstory_embedding, backstory_trace, chunk_embeddings,
           chunk_traces, W1, b1, W2, b2):
    bt = backstory_trace.reshape(1, NEURON_DIM)
    be = backstory_embedding.reshape(1, N_EMBD)
    score, idx = pl.pallas_call(
        _fused_kernel,
        grid=(STEPS,),
        in_specs=[
            pl.BlockSpec((1, NEURON_DIM), lambda i: (0, 0)),
            pl.BlockSpec((1, N_EMBD), lambda i: (0, 0)),
            pl.BlockSpec((BLK, NEURON_DIM), lambda i: (i, 0)),
            pl.BlockSpec((BLK, NEURON_DIM), lambda i: (i + STEPS, 0)),
            pl.BlockSpec((BLK, NEURON_DIM), lambda i: (i + 2 * STEPS, 0)),
            pl.BlockSpec((BLK, NEURON_DIM), lambda i: (i + 3 * STEPS, 0)),
            pl.BlockSpec(memory_space=pltpu.MemorySpace.HBM),
            pl.BlockSpec(memory_space=pltpu.MemorySpace.HBM),
            pl.BlockSpec((2 * N_EMBD + 2 * NEURON_DIM, HIDDEN),
                         lambda i: (0, 0)),
            pl.BlockSpec((1, HIDDEN), lambda i: (0, 0)),
            pl.BlockSpec((HIDDEN, 1), lambda i: (0, 0)),
            pl.BlockSpec((1, 1), lambda i: (0, 0)),
        ],
        out_specs=[
            pl.BlockSpec(memory_space=pltpu.MemorySpace.SMEM),
            pl.BlockSpec(memory_space=pltpu.MemorySpace.SMEM),
        ],
        out_shape=[
            jax.ShapeDtypeStruct((1,), jnp.float32),
            jax.ShapeDtypeStruct((1,), jnp.int32),
        ],
        scratch_shapes=[
            pltpu.VMEM((GRID, BLK), jnp.float32),
            pltpu.VMEM((GRID, 1), jnp.float32),
            pltpu.VMEM((TOP_K, N_EMBD), jnp.float32),
            pltpu.VMEM((TOP_K, NEURON_DIM), jnp.float32),
            pltpu.SMEM((TOP_K,), jnp.int32),
            pltpu.SemaphoreType.DMA,
            pltpu.SemaphoreType.DMA,
        ],
    )(bt, be, chunk_traces, chunk_traces, chunk_traces, chunk_traces,
      chunk_traces, chunk_embeddings, W1,
      b1.reshape(1, HIDDEN), W2, b2.reshape(1, 1))
    return score[0], idx[0]
